# Initial kernel scaffold; baseline (speedup 1.0000x reference)
import jax
import jax.numpy as jnp
from jax.experimental import pallas as pl


def kernel(x):
    raise NotImplementedError("write your pallas kernel here")


# speedup vs baseline: 3.7601x; 3.7601x over previous
import jax
import jax.numpy as jnp
from jax.experimental import pallas as pl
from jax.experimental.pallas import tpu as pltpu

_ROWS_PER_BLOCK = 256


def _cumsum_kern(x_ref, o_ref, carry_ref):
    i = pl.program_id(0)

    @pl.when(i == 0)
    def _zero_carry():
        carry_ref[...] = jnp.zeros_like(carry_ref)

    blk = x_ref[...]
    r = blk.shape[0]
    tri = (
        jax.lax.broadcasted_iota(jnp.int32, (r, r), 0)
        >= jax.lax.broadcasted_iota(jnp.int32, (r, r), 1)
    ).astype(jnp.float32)
    local = jnp.dot(tri, blk, preferred_element_type=jnp.float32)
    o_ref[...] = local + carry_ref[...]
    carry_ref[...] = carry_ref[...] + local[r - 1 : r, :]


def kernel(x):
    n, d = x.shape
    r = _ROWS_PER_BLOCK
    return pl.pallas_call(
        _cumsum_kern,
        grid=(n // r,),
        in_specs=[pl.BlockSpec((r, d), lambda i: (i, 0))],
        out_specs=pl.BlockSpec((r, d), lambda i: (i, 0)),
        out_shape=jax.ShapeDtypeStruct((n, d), x.dtype),
        scratch_shapes=[pltpu.VMEM((1, d), jnp.float32)],
    )(x)


# speedup vs baseline: 4.0391x; 1.0742x over previous
import jax
import jax.numpy as jnp
from jax.experimental import pallas as pl
from jax.experimental.pallas import tpu as pltpu

_ROWS_PER_BLOCK = 512


def _cumsum_kern(x_ref, o_ref, carry_ref):
    i = pl.program_id(0)

    @pl.when(i == 0)
    def _zero_carry():
        carry_ref[...] = jnp.zeros_like(carry_ref)

    blk = x_ref[...]
    r = blk.shape[0]
    tri = (
        jax.lax.broadcasted_iota(jnp.int32, (r, r), 0)
        >= jax.lax.broadcasted_iota(jnp.int32, (r, r), 1)
    ).astype(jnp.float32)
    local = jnp.dot(tri, blk, preferred_element_type=jnp.float32)
    o_ref[...] = local + carry_ref[...]
    carry_ref[...] = carry_ref[...] + local[r - 1 : r, :]


def kernel(x):
    n, d = x.shape
    r = _ROWS_PER_BLOCK
    return pl.pallas_call(
        _cumsum_kern,
        grid=(n // r,),
        in_specs=[pl.BlockSpec((r, d), lambda i: (i, 0))],
        out_specs=pl.BlockSpec((r, d), lambda i: (i, 0)),
        out_shape=jax.ShapeDtypeStruct((n, d), x.dtype),
        scratch_shapes=[pltpu.VMEM((1, d), jnp.float32)],
    )(x)
